# baseline (speedup 1.0000x reference)
import numpy as np
import jax
import jax.numpy as jnp
from jax import lax
from jax.experimental import pallas as pl
from jax.experimental.pallas import tpu as pltpu

B = 32
C = 8
SEGS = 30
SEG_LEN = 250
NFFT = 256
HOP = 128
FRAMES = 2
NFREQ = NFFT // 2 + 1
FLAT = FRAMES * NFREQ
D = 128
DFF = 512
E = 8
T = 5
H = 8
HD = D // H
NL = 2
NTOK = C * SEGS + 1
NPAD = 256

_PREC = lax.Precision.HIGHEST
_PREC_FAST = lax.Precision.DEFAULT


def _build_stft_basis():
    pos = np.arange(FRAMES)[:, None] * HOP + np.arange(NFFT)[None, :] - NFFT // 2
    j = np.abs(pos)
    j = np.where(j > SEG_LEN - 1, 2 * (SEG_LEN - 1) - j, j)
    ang = 2.0 * np.pi * np.outer(np.arange(NFFT), np.arange(NFREQ)) / NFFT
    cosb = np.cos(ang)
    sinb = np.sin(ang)
    a_cos = np.zeros((SEG_LEN, FLAT), np.float64)
    a_sin = np.zeros((SEG_LEN, FLAT), np.float64)
    for f in range(FRAMES):
        p = np.zeros((SEG_LEN, NFFT), np.float64)
        np.add.at(p, (j[f], np.arange(NFFT)), 1.0)
        a_cos[:, f::FRAMES] = p @ cosb
        a_sin[:, f::FRAMES] = p @ sinb
    return a_cos.astype(np.float32), a_sin.astype(np.float32)


_A_COS, _A_SIN = _build_stft_basis()


def _split_hi_lo(w):
    hi = w.astype(jnp.bfloat16).astype(jnp.float32)
    return hi, w - hi


def _np_split_hi_lo(w):
    import ml_dtypes
    hi = w.astype(ml_dtypes.bfloat16).astype(np.float32)
    return hi, w - hi


_A_COS_HI, _A_COS_LO = _np_split_hi_lo(_A_COS)
_A_SIN_HI, _A_SIN_LO = _np_split_hi_lo(_A_SIN)


def _dot(a, b, prec=_PREC):
    return jnp.dot(a, b, preferred_element_type=jnp.float32, precision=prec)


def _dot3(a, b_hi, b_lo):
    a_hi = a.astype(jnp.bfloat16).astype(jnp.float32)
    a_lo = a - a_hi
    return (_dot(a_hi, b_hi, _PREC_FAST) + _dot(a_hi, b_lo, _PREC_FAST)
            + _dot(a_lo, b_hi, _PREC_FAST))


def _ln(x, g, b, eps=1e-5):
    m = jnp.mean(x, axis=-1, keepdims=True)
    v = jnp.mean((x - m) ** 2, axis=-1, keepdims=True)
    return (x - m) * lax.rsqrt(v + eps) * g + b


def _gelu(x):
    return 0.5 * x * (1.0 + lax.erf(x * np.float32(1.0 / np.sqrt(2.0))))


def _attn(h, g, b, wq, wk, wv, bq, bk, bv, wo, bo, kmask):
    h2 = _ln(h, g, b)
    scale = np.float32(1.0 / np.sqrt(HD))
    q = (_dot(h2, wq, _PREC_FAST) + bq) * scale
    k = _dot(h2, wk, _PREC_FAST) + bk
    v = _dot(h2, wv, _PREC_FAST) + bv
    outs = []
    for hh in range(H):
        sl = slice(hh * HD, (hh + 1) * HD)
        s = lax.dot_general(q[:, sl], k[:, sl], (((1,), (1,)), ((), ())),
                            preferred_element_type=jnp.float32,
                            precision=_PREC_FAST)
        s = jnp.where(kmask, s, np.float32(-1e30))
        es = jnp.exp(s - jnp.max(s, axis=-1, keepdims=True))
        inv = 1.0 / jnp.sum(es, axis=-1, keepdims=True)
        outs.append(_dot(es, v[:, sl], _PREC_FAST) * inv)
    o = jnp.concatenate(outs, axis=1)
    return h + _dot(o, wo, _PREC_FAST) + bo


def _moe(h, g, b, gwh_hi, gwh_lo, gbt, ew1, eb1, ew2, eb2, uw1, ub1, uw2,
         ub2):
    h2 = _ln(h, g, b)
    logits = _dot3(h2, gwh_hi, gwh_lo) + gbt
    m1 = jnp.max(logits, axis=-1, keepdims=True)
    l2 = jnp.where(logits == m1, np.float32(-1e30), logits)
    m2 = jnp.max(l2, axis=-1, keepdims=True)
    keep = logits >= m2
    ex = jnp.where(keep, jnp.exp(logits - m1), 0.0)
    denom = jnp.sum(ex, axis=-1, keepdims=True)
    gates = ex / denom
    omega = 1.0 - 1.0 / denom
    acc = jnp.zeros((NPAD, D), jnp.float32)
    for ei in range(E):
        t1 = _gelu(_dot(h2, ew1[ei], _PREC_FAST) + eb1[ei])
        t2 = _dot(t1, ew2[ei], _PREC_FAST) + eb2[ei]
        acc = acc + gates[:, ei:ei + 1] * t2
    u = _gelu(_dot(h2, uw1, _PREC_FAST) + ub1)
    u = _dot(u, uw2, _PREC_FAST) + ub2
    return h + acc + omega * u, logits


def _fwd_body(ids_ref, x_ref, acosh_ref, acosl_ref, asinh_ref, asinl_ref,
              pwh_ref, pwl_ref, pb_ref, png_ref,
              pnb_ref, clsp_ref, pos_ref, ln1g_ref, ln1b_ref, wq_ref, wk_ref,
              wv_ref, bq_ref, bk_ref, bv_ref, wo_ref, bo_ref, ln2g_ref,
              ln2b_ref, gwhh_ref, gwhl_ref, gbt_ref, ew1_ref, eb1_ref,
              ew2_ref, eb2_ref,
              uw1_ref, ub1_ref, uw2_ref, ub2_ref, flng_ref, flnb_ref, hw_ref,
              hb_ref, logits_ref, out_ref):
    x = x_ref[0]
    x_hi = x.astype(jnp.bfloat16).astype(jnp.float32)
    x_lo = x - x_hi
    re = (_dot(x_hi, acosh_ref[...], _PREC_FAST)
          + _dot(x_hi, acosl_ref[...], _PREC_FAST)
          + _dot(x_lo, acosh_ref[...], _PREC_FAST))
    im = (_dot(x_hi, asinh_ref[...], _PREC_FAST)
          + _dot(x_hi, asinl_ref[...], _PREC_FAST)
          + _dot(x_lo, asinh_ref[...], _PREC_FAST))
    mag = jnp.sqrt(re * re + im * im)
    t = _dot3(mag, pwh_ref[...], pwl_ref[...]) + pb_ref[...]
    t = _ln(t, png_ref[...], pnb_ref[...])
    t = _gelu(t)
    t = t + pos_ref[...]
    h = jnp.concatenate(
        [clsp_ref[...], t, jnp.zeros((NPAD - NTOK, D), jnp.float32)], axis=0)

    kmask = lax.broadcasted_iota(jnp.int32, (NPAD, NPAD), 1) < NTOK
    for l in range(NL):
        h = _attn(h, ln1g_ref[l:l + 1], ln1b_ref[l:l + 1], wq_ref[l],
                  wk_ref[l], wv_ref[l], bq_ref[l:l + 1], bk_ref[l:l + 1],
                  bv_ref[l:l + 1], wo_ref[l], bo_ref[l:l + 1], kmask)
        h, logits = _moe(h, ln2g_ref[l:l + 1], ln2b_ref[l:l + 1], gwhh_ref[l],
                         gwhl_ref[l],
                         gbt_ref[0, l:l + 1], ew1_ref[l], eb1_ref[l],
                         ew2_ref[l], eb2_ref[l], uw1_ref[l],
                         ub1_ref[l, 0:1], uw2_ref[l], ub2_ref[l, 0:1])
        logits_ref[0, l] = logits

    c = _ln(h[0:1, :], flng_ref[...], flnb_ref[...])
    out_ref[0] = _dot(c, hw_ref[0]) + hb_ref[0]


def kernel(x_dict, task_ids, proj_W, proj_b, pln_g, pln_b, cls_token,
           pos_embed, ln1_g, ln1_b, Wq, Wk, Wv, bq, bk, bv, Wo, bo, ln2_g,
           ln2_b, task_embed, gate_W, gate_b, eW1, eb1, eW2, eb2, uW1, ub1,
           uW2, ub2, fln_g, fln_b, head_W, head_b):
    task_ids = task_ids.astype(jnp.int32)
    xseg = x_dict.reshape(B, C * SEGS, SEG_LEN)
    cls_pos0 = (cls_token.reshape(1, D) + pos_embed[0, 0].reshape(1, D))
    pos_rest = pos_embed[0, 1:NTOK]

    gwh_hi, gwh_lo = _split_hi_lo(gate_W[:, :D, :])
    pw_hi, pw_lo = _split_hi_lo(proj_W)
    gbt = (jnp.einsum('lte,leo->tlo', task_embed, gate_W[:, D:, :],
                      precision=lax.Precision.HIGHEST)
           + gate_b[None, :, :])

    full = lambda shp: pl.BlockSpec(shp, lambda i, ids: (0,) * len(shp))
    grid_spec = pltpu.PrefetchScalarGridSpec(
        num_scalar_prefetch=1,
        grid=(B,),
        in_specs=[
            pl.BlockSpec((1, C * SEGS, SEG_LEN), lambda i, ids: (i, 0, 0)),
            full((SEG_LEN, FLAT)), full((SEG_LEN, FLAT)),
            full((SEG_LEN, FLAT)), full((SEG_LEN, FLAT)),
            full((FLAT, D)), full((FLAT, D)),
            full((1, D)), full((1, D)), full((1, D)),
            full((1, D)), full((C * SEGS, D)),
            full((NL, D)), full((NL, D)),
            full((NL, D, D)), full((NL, D, D)), full((NL, D, D)),
            full((NL, D)), full((NL, D)), full((NL, D)),
            full((NL, D, D)), full((NL, D)),
            full((NL, D)), full((NL, D)),
            full((NL, D, E)), full((NL, D, E)),
            pl.BlockSpec((1, NL, E), lambda i, ids: (ids[i], 0, 0)),
            full((NL, E, D, DFF)), full((NL, E, DFF)),
            full((NL, E, DFF, D)), full((NL, E, D)),
            full((NL, D, DFF)), full((NL, 1, DFF)),
            full((NL, DFF, D)), full((NL, 1, D)),
            full((1, D)), full((1, D)),
            pl.BlockSpec((1, D, 2), lambda i, ids: (ids[i], 0, 0)),
            pl.BlockSpec((1, 1, 2), lambda i, ids: (ids[i], 0, 0)),
        ],
        out_specs=[
            pl.BlockSpec((1, NL, NPAD, E), lambda i, ids: (i, 0, 0, 0)),
            pl.BlockSpec((1, 1, 2), lambda i, ids: (i, 0, 0)),
        ],
    )
    logits_all, task_logits = pl.pallas_call(
        _fwd_body,
        grid_spec=grid_spec,
        out_shape=[jax.ShapeDtypeStruct((B, NL, NPAD, E), jnp.float32),
                   jax.ShapeDtypeStruct((B, 1, 2), jnp.float32)],
    )(task_ids, xseg, _A_COS_HI, _A_COS_LO, _A_SIN_HI, _A_SIN_LO,
      pw_hi, pw_lo, proj_b.reshape(1, D),
      pln_g.reshape(1, D), pln_b.reshape(1, D), cls_pos0, pos_rest,
      ln1_g, ln1_b, Wq, Wk, Wv, bq, bk, bv, Wo, bo, ln2_g, ln2_b,
      gwh_hi, gwh_lo, gbt, eW1, eb1, eW2, eb2, uW1, ub1.reshape(NL, 1, DFF),
      uW2, ub2.reshape(NL, 1, D), fln_g.reshape(1, D), fln_b.reshape(1, D),
      head_W, head_b.reshape(T, 1, 2))

    final_router = logits_all[:, :, :NTOK, :].reshape(-1, E)
    return (task_logits.reshape(B, 2), final_router)


# speedup vs baseline: 1.4080x; 1.4080x over previous
import numpy as np
import jax
import jax.numpy as jnp
from jax import lax
from jax.experimental import pallas as pl
from jax.experimental.pallas import tpu as pltpu

B = 32
C = 8
SEGS = 30
SEG_LEN = 250
NFFT = 256
HOP = 128
FRAMES = 2
NFREQ = NFFT // 2 + 1
FLAT = FRAMES * NFREQ
D = 128
DFF = 512
E = 8
T = 5
H = 8
HD = D // H
NL = 2
NTOK = C * SEGS + 1
NPAD = 256

_PREC = lax.Precision.HIGHEST
_PREC_FAST = lax.Precision.DEFAULT


def _build_stft_basis():
    pos = np.arange(FRAMES)[:, None] * HOP + np.arange(NFFT)[None, :] - NFFT // 2
    j = np.abs(pos)
    j = np.where(j > SEG_LEN - 1, 2 * (SEG_LEN - 1) - j, j)
    ang = 2.0 * np.pi * np.outer(np.arange(NFFT), np.arange(NFREQ)) / NFFT
    cosb = np.cos(ang)
    sinb = np.sin(ang)
    a_cos = np.zeros((SEG_LEN, FLAT), np.float64)
    a_sin = np.zeros((SEG_LEN, FLAT), np.float64)
    for f in range(FRAMES):
        p = np.zeros((SEG_LEN, NFFT), np.float64)
        np.add.at(p, (j[f], np.arange(NFFT)), 1.0)
        a_cos[:, f::FRAMES] = p @ cosb
        a_sin[:, f::FRAMES] = p @ sinb
    return a_cos.astype(np.float32), a_sin.astype(np.float32)


_A_COS, _A_SIN = _build_stft_basis()


def _dot(a, b, prec=_PREC):
    return jnp.dot(a, b, preferred_element_type=jnp.float32, precision=prec)


def _ln(x, g, b, eps=1e-5):
    m = jnp.mean(x, axis=-1, keepdims=True)
    v = jnp.mean((x - m) ** 2, axis=-1, keepdims=True)
    return (x - m) * lax.rsqrt(v + eps) * g + b


def _gelu(x):
    return 0.5 * x * (1.0 + lax.erf(x * np.float32(1.0 / np.sqrt(2.0))))


def _attn(h, g, b, wq, wk, wv, bq, bk, bv, wo, bo, kmask):
    h2 = _ln(h, g, b)
    scale = np.float32(1.0 / np.sqrt(HD))
    q = (_dot(h2, wq, _PREC_FAST) + bq) * scale
    k = _dot(h2, wk, _PREC_FAST) + bk
    v = _dot(h2, wv, _PREC_FAST) + bv
    ones_col = jnp.ones((NPAD, 1), jnp.float32)
    outs = []
    for hh in range(H):
        sl = slice(hh * HD, (hh + 1) * HD)
        s = lax.dot_general(q[:, sl], k[:, sl], (((1,), (1,)), ((), ())),
                            preferred_element_type=jnp.float32,
                            precision=_PREC_FAST)
        es = jnp.exp(jnp.where(kmask, jnp.minimum(s, np.float32(60.0)),
                               np.float32(-1e30)))
        ve = jnp.concatenate([v[:, sl], ones_col], axis=1)
        oh = _dot(es, ve, _PREC_FAST)
        outs.append(oh[:, :HD] * (1.0 / oh[:, HD:HD + 1]))
    o = jnp.concatenate(outs, axis=1)
    return h + _dot(o, wo, _PREC_FAST) + bo


def _moe(h, g, b, gwh, gbt, ew1, eb1, ew2, eb2, uw1, ub1, uw2, ub2):
    h2 = _ln(h, g, b)
    logits = _dot(h2, gwh) + gbt
    m1 = jnp.max(logits, axis=-1, keepdims=True)
    l2 = jnp.where(logits == m1, np.float32(-1e30), logits)
    m2 = jnp.max(l2, axis=-1, keepdims=True)
    keep = logits >= m2
    ex = jnp.where(keep, jnp.exp(logits - m1), 0.0)
    denom = jnp.sum(ex, axis=-1, keepdims=True)
    gates = ex / denom
    omega = 1.0 - 1.0 / denom
    acc = jnp.zeros((NPAD, D), jnp.float32)
    for ei in range(E):
        t1 = _gelu(_dot(h2, ew1[ei], _PREC_FAST) + eb1[ei])
        t2 = _dot(t1, ew2[ei], _PREC_FAST) + eb2[ei]
        acc = acc + gates[:, ei:ei + 1] * t2
    u = _gelu(_dot(h2, uw1, _PREC_FAST) + ub1)
    u = _dot(u, uw2, _PREC_FAST) + ub2
    return h + acc + omega * u, logits


def _fwd_body(ids_ref, x_ref, acos_ref, asin_ref, pw_ref, pb_ref, png_ref,
              pnb_ref, clsp_ref, pos_ref, ln1g_ref, ln1b_ref, wq_ref, wk_ref,
              wv_ref, bq_ref, bk_ref, bv_ref, wo_ref, bo_ref, ln2g_ref,
              ln2b_ref, gwh_ref, gbt_ref, ew1_ref, eb1_ref,
              ew2_ref, eb2_ref,
              uw1_ref, ub1_ref, uw2_ref, ub2_ref, flng_ref, flnb_ref,
              hw_ref, hb_ref, logits_ref, out_ref):
    kmask = lax.broadcasted_iota(jnp.int32, (NPAD, NPAD), 1) < NTOK
    x = x_ref[0]
    re = _dot(x, acos_ref[...])
    im = _dot(x, asin_ref[...])
    mag = jnp.sqrt(re * re + im * im)
    t = _dot(mag, pw_ref[...]) + pb_ref[...]
    t = _ln(t, png_ref[...], pnb_ref[...])
    t = _gelu(t)
    t = t + pos_ref[...]
    h = jnp.concatenate(
        [clsp_ref[...], t, jnp.zeros((NPAD - NTOK, D), jnp.float32)], axis=0)

    for l in range(NL):
        h = _attn(h, ln1g_ref[l:l + 1], ln1b_ref[l:l + 1], wq_ref[l],
                  wk_ref[l], wv_ref[l], bq_ref[l:l + 1], bk_ref[l:l + 1],
                  bv_ref[l:l + 1], wo_ref[l], bo_ref[l:l + 1], kmask)
        h, logits = _moe(h, ln2g_ref[l:l + 1], ln2b_ref[l:l + 1],
                         gwh_ref[l], gbt_ref[0, l:l + 1], ew1_ref[l],
                         eb1_ref[l],
                         ew2_ref[l], eb2_ref[l], uw1_ref[l],
                         ub1_ref[l, 0:1], uw2_ref[l], ub2_ref[l, 0:1])
        logits_ref[0, l] = logits

    c = _ln(h[0:1, :], flng_ref[...], flnb_ref[...])
    out_ref[0] = _dot(c, hw_ref[0]) + hb_ref[0]


def kernel(x_dict, task_ids, proj_W, proj_b, pln_g, pln_b, cls_token,
           pos_embed, ln1_g, ln1_b, Wq, Wk, Wv, bq, bk, bv, Wo, bo, ln2_g,
           ln2_b, task_embed, gate_W, gate_b, eW1, eb1, eW2, eb2, uW1, ub1,
           uW2, ub2, fln_g, fln_b, head_W, head_b):
    task_ids = task_ids.astype(jnp.int32)
    xseg = x_dict.reshape(B, C * SEGS, SEG_LEN)
    cls_pos0 = (cls_token.reshape(1, D) + pos_embed[0, 0].reshape(1, D))
    pos_rest = pos_embed[0, 1:NTOK]

    gwh = gate_W[:, :D, :]
    gbt = (jnp.einsum('lte,leo->tlo', task_embed, gate_W[:, D:, :],
                      precision=lax.Precision.HIGHEST)
           + gate_b[None, :, :])

    full = lambda shp: pl.BlockSpec(shp, lambda i, ids: (0,) * len(shp))
    grid_spec = pltpu.PrefetchScalarGridSpec(
        num_scalar_prefetch=1,
        grid=(B,),
        in_specs=[
            pl.BlockSpec((1, C * SEGS, SEG_LEN), lambda i, ids: (i, 0, 0)),
            full((SEG_LEN, FLAT)), full((SEG_LEN, FLAT)),
            full((FLAT, D)), full((1, D)), full((1, D)), full((1, D)),
            full((1, D)), full((C * SEGS, D)),
            full((NL, D)), full((NL, D)),
            full((NL, D, D)), full((NL, D, D)), full((NL, D, D)),
            full((NL, D)), full((NL, D)), full((NL, D)),
            full((NL, D, D)), full((NL, D)),
            full((NL, D)), full((NL, D)),
            full((NL, D, E)),
            pl.BlockSpec((1, NL, E), lambda i, ids: (ids[i], 0, 0)),
            full((NL, E, D, DFF)), full((NL, E, DFF)),
            full((NL, E, DFF, D)), full((NL, E, D)),
            full((NL, D, DFF)), full((NL, 1, DFF)),
            full((NL, DFF, D)), full((NL, 1, D)),
            full((1, D)), full((1, D)),
            pl.BlockSpec((1, D, 2), lambda i, ids: (ids[i], 0, 0)),
            pl.BlockSpec((1, 1, 2), lambda i, ids: (ids[i], 0, 0)),
        ],
        out_specs=[
            pl.BlockSpec((1, NL, NPAD, E), lambda i, ids: (i, 0, 0, 0)),
            pl.BlockSpec((1, 1, 2), lambda i, ids: (i, 0, 0)),
        ],
    )
    logits_all, task_logits = pl.pallas_call(
        _fwd_body,
        grid_spec=grid_spec,
        out_shape=[jax.ShapeDtypeStruct((B, NL, NPAD, E), jnp.float32),
                   jax.ShapeDtypeStruct((B, 1, 2), jnp.float32)],
    )(task_ids, xseg, _A_COS, _A_SIN, proj_W, proj_b.reshape(1, D),
      pln_g.reshape(1, D), pln_b.reshape(1, D), cls_pos0, pos_rest,
      ln1_g, ln1_b, Wq, Wk, Wv, bq, bk, bv, Wo, bo, ln2_g, ln2_b,
      gwh, gbt, eW1, eb1, eW2, eb2, uW1, ub1.reshape(NL, 1, DFF),
      uW2, ub2.reshape(NL, 1, D), fln_g.reshape(1, D), fln_b.reshape(1, D),
      head_W, head_b.reshape(T, 1, 2))

    final_router = logits_all[:, :, :NTOK, :].reshape(-1, E)
    return (task_logits.reshape(B, 2), final_router)


# speedup vs baseline: 1.4946x; 1.0615x over previous
import numpy as np
import jax
import jax.numpy as jnp
from jax import lax
from jax.experimental import pallas as pl
from jax.experimental.pallas import tpu as pltpu

B = 32
C = 8
SEGS = 30
SEG_LEN = 250
NFFT = 256
HOP = 128
FRAMES = 2
NFREQ = NFFT // 2 + 1
FLAT = FRAMES * NFREQ
D = 128
DFF = 512
E = 8
T = 5
H = 8
HD = D // H
NL = 2
NTOK = C * SEGS + 1
NPAD = 256

_PREC = lax.Precision.HIGHEST
_PREC_FAST = lax.Precision.DEFAULT


def _build_stft_basis():
    pos = np.arange(FRAMES)[:, None] * HOP + np.arange(NFFT)[None, :] - NFFT // 2
    j = np.abs(pos)
    j = np.where(j > SEG_LEN - 1, 2 * (SEG_LEN - 1) - j, j)
    ang = 2.0 * np.pi * np.outer(np.arange(NFFT), np.arange(NFREQ)) / NFFT
    cosb = np.cos(ang)
    sinb = np.sin(ang)
    a_cos = np.zeros((SEG_LEN, FLAT), np.float64)
    a_sin = np.zeros((SEG_LEN, FLAT), np.float64)
    for f in range(FRAMES):
        p = np.zeros((SEG_LEN, NFFT), np.float64)
        np.add.at(p, (j[f], np.arange(NFFT)), 1.0)
        a_cos[:, f::FRAMES] = p @ cosb
        a_sin[:, f::FRAMES] = p @ sinb
    return a_cos.astype(np.float32), a_sin.astype(np.float32)


_A_COS, _A_SIN = _build_stft_basis()


def _np_split_hi_lo(w):
    import ml_dtypes
    hi = w.astype(ml_dtypes.bfloat16).astype(np.float32)
    return hi, (w - hi).astype(np.float32)


_A_COS_HI, _A_COS_LO = _np_split_hi_lo(_A_COS)
_A_SIN_HI, _A_SIN_LO = _np_split_hi_lo(_A_SIN)


def _dot(a, b, prec=_PREC):
    return jnp.dot(a, b, preferred_element_type=jnp.float32, precision=prec)


def _dot3(a, b_hi, b_lo):
    a_hi = a.astype(jnp.bfloat16).astype(jnp.float32)
    a_lo = a - a_hi
    return (_dot(a_hi, b_hi, _PREC_FAST) + _dot(a_hi, b_lo, _PREC_FAST)
            + _dot(a_lo, b_hi, _PREC_FAST))


def _split_hi_lo(w):
    hi = w.astype(jnp.bfloat16).astype(jnp.float32)
    return hi, w - hi


def _ln(x, g, b, eps=1e-5):
    m = jnp.mean(x, axis=-1, keepdims=True)
    v = jnp.mean((x - m) ** 2, axis=-1, keepdims=True)
    return (x - m) * lax.rsqrt(v + eps) * g + b


def _gelu(x):
    return 0.5 * x * (1.0 + lax.erf(x * np.float32(1.0 / np.sqrt(2.0))))


def _attn(h, g, b, wq, wk, wv, bq, bk, bv, wo, bo, kmask):
    h2 = _ln(h, g, b)
    scale = np.float32(1.0 / np.sqrt(HD))
    q = (_dot(h2, wq, _PREC_FAST) + bq) * scale
    k = _dot(h2, wk, _PREC_FAST) + bk
    v = _dot(h2, wv, _PREC_FAST) + bv
    ones_col = jnp.ones((NPAD, 1), jnp.float32)
    outs = []
    for hh in range(H):
        sl = slice(hh * HD, (hh + 1) * HD)
        s = lax.dot_general(q[:, sl], k[:, sl], (((1,), (1,)), ((), ())),
                            preferred_element_type=jnp.float32,
                            precision=_PREC_FAST)
        es = jnp.exp(jnp.where(kmask, jnp.minimum(s, np.float32(60.0)),
                               np.float32(-1e30)))
        ve = jnp.concatenate([v[:, sl], ones_col], axis=1)
        oh = _dot(es, ve, _PREC_FAST)
        outs.append(oh[:, :HD] * (1.0 / oh[:, HD:HD + 1]))
    o = jnp.concatenate(outs, axis=1)
    return h + _dot(o, wo, _PREC_FAST) + bo


def _moe(h, g, b, gwh_hi, gwh_lo, gbt, ew1, eb1, ew2, eb2, uw1, ub1, uw2,
         ub2):
    h2 = _ln(h, g, b)
    logits = _dot3(h2, gwh_hi, gwh_lo) + gbt
    m1 = jnp.max(logits, axis=-1, keepdims=True)
    l2 = jnp.where(logits == m1, np.float32(-1e30), logits)
    m2 = jnp.max(l2, axis=-1, keepdims=True)
    keep = logits >= m2
    ex = jnp.where(keep, jnp.exp(logits - m1), 0.0)
    denom = jnp.sum(ex, axis=-1, keepdims=True)
    acc = jnp.zeros((NPAD, D), jnp.float32)
    for ei in range(E):
        t1 = _gelu(_dot(h2, ew1[ei], _PREC_FAST) + eb1[ei])
        t2 = _dot(t1, ew2[ei], _PREC_FAST) + eb2[ei]
        acc = acc + ex[:, ei:ei + 1] * t2
    u = _gelu(_dot(h2, uw1, _PREC_FAST) + ub1)
    u = _dot(u, uw2, _PREC_FAST) + ub2
    invd = 1.0 / denom
    return h + acc * invd + (1.0 - invd) * u, logits


def _fwd_body(ids_ref, x_ref, acosh_ref, acosl_ref, asinh_ref, asinl_ref, pwh_ref, pwl_ref, pb_ref, png_ref,
              pnb_ref, clsp_ref, pos_ref, ln1g_ref, ln1b_ref, wq_ref, wk_ref,
              wv_ref, bq_ref, bk_ref, bv_ref, wo_ref, bo_ref, ln2g_ref,
              ln2b_ref, gwhh_ref, gwhl_ref, gbt_ref, ew1_ref, eb1_ref,
              ew2_ref, eb2_ref,
              uw1_ref, ub1_ref, uw2_ref, ub2_ref, flng_ref, flnb_ref,
              hw_ref, hb_ref, logits_ref, out_ref):
    kmask = lax.broadcasted_iota(jnp.int32, (NPAD, NPAD), 1) < NTOK
    x = x_ref[0]
    x_hi = x.astype(jnp.bfloat16).astype(jnp.float32)
    x_lo = x - x_hi
    re = (_dot(x_hi, acosh_ref[...], _PREC_FAST)
          + _dot(x_hi, acosl_ref[...], _PREC_FAST)
          + _dot(x_lo, acosh_ref[...], _PREC_FAST))
    im = (_dot(x_hi, asinh_ref[...], _PREC_FAST)
          + _dot(x_hi, asinl_ref[...], _PREC_FAST)
          + _dot(x_lo, asinh_ref[...], _PREC_FAST))
    mag = jnp.sqrt(re * re + im * im)
    t = _dot3(mag, pwh_ref[...], pwl_ref[...]) + pb_ref[...]
    t = _ln(t, png_ref[...], pnb_ref[...])
    t = _gelu(t)
    t = t + pos_ref[...]
    h = jnp.concatenate(
        [clsp_ref[...], t, jnp.zeros((NPAD - NTOK, D), jnp.float32)], axis=0)

    for l in range(NL):
        h = _attn(h, ln1g_ref[l:l + 1], ln1b_ref[l:l + 1], wq_ref[l],
                  wk_ref[l], wv_ref[l], bq_ref[l:l + 1], bk_ref[l:l + 1],
                  bv_ref[l:l + 1], wo_ref[l], bo_ref[l:l + 1], kmask)
        h, logits = _moe(h, ln2g_ref[l:l + 1], ln2b_ref[l:l + 1],
                         gwhh_ref[l], gwhl_ref[l],
                         gbt_ref[0, l:l + 1], ew1_ref[l],
                         eb1_ref[l],
                         ew2_ref[l], eb2_ref[l], uw1_ref[l],
                         ub1_ref[l, 0:1], uw2_ref[l], ub2_ref[l, 0:1])
        logits_ref[0, l] = logits

    c = _ln(h[0:1, :], flng_ref[...], flnb_ref[...])
    out_ref[0] = _dot(c, hw_ref[0]) + hb_ref[0]


def kernel(x_dict, task_ids, proj_W, proj_b, pln_g, pln_b, cls_token,
           pos_embed, ln1_g, ln1_b, Wq, Wk, Wv, bq, bk, bv, Wo, bo, ln2_g,
           ln2_b, task_embed, gate_W, gate_b, eW1, eb1, eW2, eb2, uW1, ub1,
           uW2, ub2, fln_g, fln_b, head_W, head_b):
    task_ids = task_ids.astype(jnp.int32)
    xseg = x_dict.reshape(B, C * SEGS, SEG_LEN)
    cls_pos0 = (cls_token.reshape(1, D) + pos_embed[0, 0].reshape(1, D))
    pw_hi, pw_lo = _split_hi_lo(proj_W)
    pos_rest = pos_embed[0, 1:NTOK]

    gwh_hi, gwh_lo = _split_hi_lo(gate_W[:, :D, :])
    gbt = (jnp.einsum('lte,leo->tlo', task_embed, gate_W[:, D:, :],
                      precision=lax.Precision.HIGHEST)
           + gate_b[None, :, :])

    full = lambda shp: pl.BlockSpec(shp, lambda i, ids: (0,) * len(shp))
    grid_spec = pltpu.PrefetchScalarGridSpec(
        num_scalar_prefetch=1,
        grid=(B,),
        in_specs=[
            pl.BlockSpec((1, C * SEGS, SEG_LEN), lambda i, ids: (i, 0, 0)),
            full((SEG_LEN, FLAT)), full((SEG_LEN, FLAT)),
            full((SEG_LEN, FLAT)), full((SEG_LEN, FLAT)),
            full((FLAT, D)), full((FLAT, D)), full((1, D)), full((1, D)), full((1, D)),
            full((1, D)), full((C * SEGS, D)),
            full((NL, D)), full((NL, D)),
            full((NL, D, D)), full((NL, D, D)), full((NL, D, D)),
            full((NL, D)), full((NL, D)), full((NL, D)),
            full((NL, D, D)), full((NL, D)),
            full((NL, D)), full((NL, D)),
            full((NL, D, E)), full((NL, D, E)),
            pl.BlockSpec((1, NL, E), lambda i, ids: (ids[i], 0, 0)),
            full((NL, E, D, DFF)), full((NL, E, DFF)),
            full((NL, E, DFF, D)), full((NL, E, D)),
            full((NL, D, DFF)), full((NL, 1, DFF)),
            full((NL, DFF, D)), full((NL, 1, D)),
            full((1, D)), full((1, D)),
            pl.BlockSpec((1, D, 2), lambda i, ids: (ids[i], 0, 0)),
            pl.BlockSpec((1, 1, 2), lambda i, ids: (ids[i], 0, 0)),
        ],
        out_specs=[
            pl.BlockSpec((1, NL, NPAD, E), lambda i, ids: (i, 0, 0, 0)),
            pl.BlockSpec((1, 1, 2), lambda i, ids: (i, 0, 0)),
        ],
    )
    logits_all, task_logits = pl.pallas_call(
        _fwd_body,
        grid_spec=grid_spec,
        out_shape=[jax.ShapeDtypeStruct((B, NL, NPAD, E), jnp.float32),
                   jax.ShapeDtypeStruct((B, 1, 2), jnp.float32)],
    )(task_ids, xseg, _A_COS_HI, _A_COS_LO, _A_SIN_HI, _A_SIN_LO, pw_hi, pw_lo, proj_b.reshape(1, D),
      pln_g.reshape(1, D), pln_b.reshape(1, D), cls_pos0, pos_rest,
      ln1_g, ln1_b, Wq, Wk, Wv, bq, bk, bv, Wo, bo, ln2_g, ln2_b,
      gwh_hi, gwh_lo, gbt, eW1, eb1, eW2, eb2, uW1, ub1.reshape(NL, 1, DFF),
      uW2, ub2.reshape(NL, 1, D), fln_g.reshape(1, D), fln_b.reshape(1, D),
      head_W, head_b.reshape(T, 1, 2))

    final_router = logits_all[:, :, :NTOK, :].reshape(-1, E)
    return (task_logits.reshape(B, 2), final_router)


# speedup vs baseline: 1.5186x; 1.0161x over previous
import numpy as np
import jax
import jax.numpy as jnp
from jax import lax
from jax.experimental import pallas as pl
from jax.experimental.pallas import tpu as pltpu

B = 32
C = 8
SEGS = 30
SEG_LEN = 250
NFFT = 256
HOP = 128
FRAMES = 2
NFREQ = NFFT // 2 + 1
FLAT = FRAMES * NFREQ
D = 128
DFF = 512
E = 8
T = 5
H = 8
HD = D // H
NL = 2
NTOK = C * SEGS + 1
NPAD = 256

_PREC = lax.Precision.HIGHEST
_PREC_FAST = lax.Precision.DEFAULT


def _build_stft_basis():
    pos = np.arange(FRAMES)[:, None] * HOP + np.arange(NFFT)[None, :] - NFFT // 2
    j = np.abs(pos)
    j = np.where(j > SEG_LEN - 1, 2 * (SEG_LEN - 1) - j, j)
    ang = 2.0 * np.pi * np.outer(np.arange(NFFT), np.arange(NFREQ)) / NFFT
    cosb = np.cos(ang)
    sinb = np.sin(ang)
    a_cos = np.zeros((SEG_LEN, FLAT), np.float64)
    a_sin = np.zeros((SEG_LEN, FLAT), np.float64)
    for f in range(FRAMES):
        p = np.zeros((SEG_LEN, NFFT), np.float64)
        np.add.at(p, (j[f], np.arange(NFFT)), 1.0)
        a_cos[:, f::FRAMES] = p @ cosb
        a_sin[:, f::FRAMES] = p @ sinb
    return a_cos.astype(np.float32), a_sin.astype(np.float32)


_A_COS, _A_SIN = _build_stft_basis()


def _np_split_hi_lo(w):
    hi = (w.view(np.uint32) & np.uint32(0xFFFF0000)).view(np.float32)
    return hi, (w - hi).astype(np.float32)


_A_COS_HI, _A_COS_LO = _np_split_hi_lo(_A_COS)
_A_SIN_HI, _A_SIN_LO = _np_split_hi_lo(_A_SIN)


def _dot(a, b, prec=_PREC):
    return jnp.dot(a, b, preferred_element_type=jnp.float32, precision=prec)


def _trunc_bf16(x):
    bits = lax.bitcast_convert_type(x, jnp.uint32) & np.uint32(0xFFFF0000)
    return lax.bitcast_convert_type(bits, jnp.float32)


def _dot3(a, b_hi, b_lo):
    a_hi = _trunc_bf16(a)
    a_lo = a - a_hi
    return (_dot(a_hi, b_hi, _PREC_FAST) + _dot(a_hi, b_lo, _PREC_FAST)
            + _dot(a_lo, b_hi, _PREC_FAST))


def _split_hi_lo(w):
    hi = _trunc_bf16(w)
    return hi, w - hi


def _ln(x, g, b, eps=1e-5):
    m = jnp.mean(x, axis=-1, keepdims=True)
    v = jnp.mean((x - m) ** 2, axis=-1, keepdims=True)
    return (x - m) * lax.rsqrt(v + eps) * g + b


def _gelu(x):
    return 0.5 * x * (1.0 + lax.erf(x * np.float32(1.0 / np.sqrt(2.0))))


def _attn(h, g, b, wq, wk, wv, bq, bk, bv, wo, bo, kmask):
    h2 = _ln(h, g, b)
    scale = np.float32(1.0 / np.sqrt(HD))
    q = (_dot(h2, wq, _PREC_FAST) + bq) * scale
    k = _dot(h2, wk, _PREC_FAST) + bk
    v = _dot(h2, wv, _PREC_FAST) + bv
    ones_col = jnp.ones((NPAD, 1), jnp.float32)
    outs = []
    for hh in range(H):
        sl = slice(hh * HD, (hh + 1) * HD)
        s = lax.dot_general(q[:, sl], k[:, sl], (((1,), (1,)), ((), ())),
                            preferred_element_type=jnp.float32,
                            precision=_PREC_FAST)
        es = jnp.exp(jnp.where(kmask, jnp.minimum(s, np.float32(60.0)),
                               np.float32(-1e30)))
        ve = jnp.concatenate([v[:, sl], ones_col], axis=1)
        oh = _dot(es, ve, _PREC_FAST)
        outs.append(oh[:, :HD] * (1.0 / oh[:, HD:HD + 1]))
    o = jnp.concatenate(outs, axis=1)
    return h + _dot(o, wo, _PREC_FAST) + bo


def _moe(h, g, b, gwh_hi, gwh_lo, gbt, ew1, eb1, ew2, eb2, uw1, ub1, uw2,
         ub2):
    h2 = _ln(h, g, b)
    logits = _dot3(h2, gwh_hi, gwh_lo) + gbt
    m1 = jnp.max(logits, axis=-1, keepdims=True)
    l2 = jnp.where(logits == m1, np.float32(-1e30), logits)
    m2 = jnp.max(l2, axis=-1, keepdims=True)
    keep = logits >= m2
    ex = jnp.where(keep, jnp.exp(logits - m1), 0.0)
    denom = jnp.sum(ex, axis=-1, keepdims=True)
    gates = ex / denom
    omega = 1.0 - 1.0 / denom
    acc = jnp.zeros((NPAD, D), jnp.float32)
    for ei in range(E):
        t1 = _gelu(_dot(h2, ew1[ei], _PREC_FAST) + eb1[ei])
        t2 = _dot(t1, ew2[ei], _PREC_FAST) + eb2[ei]
        acc = acc + gates[:, ei:ei + 1] * t2
    u = _gelu(_dot(h2, uw1, _PREC_FAST) + ub1)
    u = _dot(u, uw2, _PREC_FAST) + ub2
    return h + acc + omega * u, logits


def _fwd_body(ids_ref, x_ref, acosh_ref, acosl_ref, asinh_ref, asinl_ref, pwh_ref, pwl_ref, pb_ref, png_ref,
              pnb_ref, clsp_ref, pos_ref, ln1g_ref, ln1b_ref, wq_ref, wk_ref,
              wv_ref, bq_ref, bk_ref, bv_ref, wo_ref, bo_ref, ln2g_ref,
              ln2b_ref, gwhh_ref, gwhl_ref, gbt_ref, ew1_ref, eb1_ref,
              ew2_ref, eb2_ref,
              uw1_ref, ub1_ref, uw2_ref, ub2_ref, flng_ref, flnb_ref,
              hw_ref, hb_ref, logits_ref, out_ref):
    kmask = lax.broadcasted_iota(jnp.int32, (NPAD, NPAD), 1) < NTOK
    x = x_ref[0]
    x_hi = _trunc_bf16(x)
    x_lo = x - x_hi
    re = (_dot(x_hi, acosh_ref[...], _PREC_FAST)
          + _dot(x_hi, acosl_ref[...], _PREC_FAST)
          + _dot(x_lo, acosh_ref[...], _PREC_FAST))
    im = (_dot(x_hi, asinh_ref[...], _PREC_FAST)
          + _dot(x_hi, asinl_ref[...], _PREC_FAST)
          + _dot(x_lo, asinh_ref[...], _PREC_FAST))
    mag = jnp.sqrt(re * re + im * im)
    t = _dot3(mag, pwh_ref[...], pwl_ref[...]) + pb_ref[...]
    t = _ln(t, png_ref[...], pnb_ref[...])
    t = _gelu(t)
    t = t + pos_ref[...]
    h = jnp.concatenate(
        [clsp_ref[...], t, jnp.zeros((NPAD - NTOK, D), jnp.float32)], axis=0)

    for l in range(NL):
        h = _attn(h, ln1g_ref[l:l + 1], ln1b_ref[l:l + 1], wq_ref[l],
                  wk_ref[l], wv_ref[l], bq_ref[l:l + 1], bk_ref[l:l + 1],
                  bv_ref[l:l + 1], wo_ref[l], bo_ref[l:l + 1], kmask)
        h, logits = _moe(h, ln2g_ref[l:l + 1], ln2b_ref[l:l + 1],
                         gwhh_ref[l], gwhl_ref[l],
                         gbt_ref[0, l:l + 1], ew1_ref[l],
                         eb1_ref[l],
                         ew2_ref[l], eb2_ref[l], uw1_ref[l],
                         ub1_ref[l, 0:1], uw2_ref[l], ub2_ref[l, 0:1])
        logits_ref[0, l] = logits

    c = _ln(h[0:1, :], flng_ref[...], flnb_ref[...])
    out_ref[0] = _dot(c, hw_ref[0]) + hb_ref[0]


def kernel(x_dict, task_ids, proj_W, proj_b, pln_g, pln_b, cls_token,
           pos_embed, ln1_g, ln1_b, Wq, Wk, Wv, bq, bk, bv, Wo, bo, ln2_g,
           ln2_b, task_embed, gate_W, gate_b, eW1, eb1, eW2, eb2, uW1, ub1,
           uW2, ub2, fln_g, fln_b, head_W, head_b):
    task_ids = task_ids.astype(jnp.int32)
    xseg = x_dict.reshape(B, C * SEGS, SEG_LEN)
    cls_pos0 = (cls_token.reshape(1, D) + pos_embed[0, 0].reshape(1, D))
    pw_hi, pw_lo = _split_hi_lo(proj_W)
    pos_rest = pos_embed[0, 1:NTOK]

    gwh_hi, gwh_lo = _split_hi_lo(gate_W[:, :D, :])
    gbt = (jnp.einsum('lte,leo->tlo', task_embed, gate_W[:, D:, :],
                      precision=lax.Precision.HIGHEST)
           + gate_b[None, :, :])

    full = lambda shp: pl.BlockSpec(shp, lambda i, ids: (0,) * len(shp))
    grid_spec = pltpu.PrefetchScalarGridSpec(
        num_scalar_prefetch=1,
        grid=(B,),
        in_specs=[
            pl.BlockSpec((1, C * SEGS, SEG_LEN), lambda i, ids: (i, 0, 0)),
            full((SEG_LEN, FLAT)), full((SEG_LEN, FLAT)),
            full((SEG_LEN, FLAT)), full((SEG_LEN, FLAT)),
            full((FLAT, D)), full((FLAT, D)), full((1, D)), full((1, D)), full((1, D)),
            full((1, D)), full((C * SEGS, D)),
            full((NL, D)), full((NL, D)),
            full((NL, D, D)), full((NL, D, D)), full((NL, D, D)),
            full((NL, D)), full((NL, D)), full((NL, D)),
            full((NL, D, D)), full((NL, D)),
            full((NL, D)), full((NL, D)),
            full((NL, D, E)), full((NL, D, E)),
            pl.BlockSpec((1, NL, E), lambda i, ids: (ids[i], 0, 0)),
            full((NL, E, D, DFF)), full((NL, E, DFF)),
            full((NL, E, DFF, D)), full((NL, E, D)),
            full((NL, D, DFF)), full((NL, 1, DFF)),
            full((NL, DFF, D)), full((NL, 1, D)),
            full((1, D)), full((1, D)),
            pl.BlockSpec((1, D, 2), lambda i, ids: (ids[i], 0, 0)),
            pl.BlockSpec((1, 1, 2), lambda i, ids: (ids[i], 0, 0)),
        ],
        out_specs=[
            pl.BlockSpec((1, NL, NPAD, E), lambda i, ids: (i, 0, 0, 0)),
            pl.BlockSpec((1, 1, 2), lambda i, ids: (i, 0, 0)),
        ],
    )
    logits_all, task_logits = pl.pallas_call(
        _fwd_body,
        grid_spec=grid_spec,
        out_shape=[jax.ShapeDtypeStruct((B, NL, NPAD, E), jnp.float32),
                   jax.ShapeDtypeStruct((B, 1, 2), jnp.float32)],
    )(task_ids, xseg, _A_COS_HI, _A_COS_LO, _A_SIN_HI, _A_SIN_LO, pw_hi, pw_lo, proj_b.reshape(1, D),
      pln_g.reshape(1, D), pln_b.reshape(1, D), cls_pos0, pos_rest,
      ln1_g, ln1_b, Wq, Wk, Wv, bq, bk, bv, Wo, bo, ln2_g, ln2_b,
      gwh_hi, gwh_lo, gbt, eW1, eb1, eW2, eb2, uW1, ub1.reshape(NL, 1, DFF),
      uW2, ub2.reshape(NL, 1, D), fln_g.reshape(1, D), fln_b.reshape(1, D),
      head_W, head_b.reshape(T, 1, 2))

    final_router = logits_all[:, :, :NTOK, :].reshape(-1, E)
    return (task_logits.reshape(B, 2), final_router)


# speedup vs baseline: 1.5652x; 1.0307x over previous
import numpy as np
import jax
import jax.numpy as jnp
from jax import lax
from jax.experimental import pallas as pl
from jax.experimental.pallas import tpu as pltpu

B = 32
C = 8
SEGS = 30
SEG_LEN = 250
NFFT = 256
HOP = 128
FRAMES = 2
NFREQ = NFFT // 2 + 1
FLAT = FRAMES * NFREQ
D = 128
DFF = 512
E = 8
T = 5
H = 8
HD = D // H
NL = 2
NTOK = C * SEGS + 1
NPAD = 256

_PREC = lax.Precision.HIGHEST
_PREC_FAST = lax.Precision.DEFAULT


def _build_stft_basis():
    pos = np.arange(FRAMES)[:, None] * HOP + np.arange(NFFT)[None, :] - NFFT // 2
    j = np.abs(pos)
    j = np.where(j > SEG_LEN - 1, 2 * (SEG_LEN - 1) - j, j)
    ang = 2.0 * np.pi * np.outer(np.arange(NFFT), np.arange(NFREQ)) / NFFT
    cosb = np.cos(ang)
    sinb = np.sin(ang)
    a_cos = np.zeros((SEG_LEN, FLAT), np.float64)
    a_sin = np.zeros((SEG_LEN, FLAT), np.float64)
    for f in range(FRAMES):
        p = np.zeros((SEG_LEN, NFFT), np.float64)
        np.add.at(p, (j[f], np.arange(NFFT)), 1.0)
        a_cos[:, f::FRAMES] = p @ cosb
        a_sin[:, f::FRAMES] = p @ sinb
    return a_cos.astype(np.float32), a_sin.astype(np.float32)


_A_COS, _A_SIN = _build_stft_basis()


def _np_split_hi_lo(w):
    hi = (w.view(np.uint32) & np.uint32(0xFFFF0000)).view(np.float32)
    return hi, (w - hi).astype(np.float32)


_A_COS_HI, _A_COS_LO = _np_split_hi_lo(_A_COS)
_A_SIN_HI, _A_SIN_LO = _np_split_hi_lo(_A_SIN)


def _dot(a, b, prec=_PREC):
    return jnp.dot(a, b, preferred_element_type=jnp.float32, precision=prec)


def _trunc_bf16(x):
    bits = lax.bitcast_convert_type(x, jnp.uint32) & np.uint32(0xFFFF0000)
    return lax.bitcast_convert_type(bits, jnp.float32)


def _dot3(a, b_hi, b_lo):
    a_hi = _trunc_bf16(a)
    a_lo = a - a_hi
    return (_dot(a_hi, b_hi, _PREC_FAST) + _dot(a_hi, b_lo, _PREC_FAST)
            + _dot(a_lo, b_hi, _PREC_FAST))


def _split_hi_lo(w):
    hi = _trunc_bf16(w)
    return hi, w - hi


def _ln(x, g, b, eps=1e-5):
    m = jnp.mean(x, axis=-1, keepdims=True)
    v = jnp.mean((x - m) ** 2, axis=-1, keepdims=True)
    return (x - m) * lax.rsqrt(v + eps) * g + b


def _gelu(x):
    return 0.5 * x * (1.0 + lax.erf(x * np.float32(1.0 / np.sqrt(2.0))))


def _attn(h, g, b, wq, wk, wv, bq, bk, bv, wo, bo, kmask):
    h2 = _ln(h, g, b)
    scale = np.float32(1.0 / np.sqrt(HD))
    q = (_dot(h2, wq, _PREC_FAST) + bq) * scale
    k = _dot(h2, wk, _PREC_FAST) + bk
    v = _dot(h2, wv, _PREC_FAST) + bv
    ones_col = jnp.ones((NPAD, 1), jnp.float32)
    outs = []
    for hh in range(H):
        sl = slice(hh * HD, (hh + 1) * HD)
        s = lax.dot_general(q[:, sl], k[:, sl], (((1,), (1,)), ((), ())),
                            preferred_element_type=jnp.float32,
                            precision=_PREC_FAST)
        es = jnp.exp(jnp.where(kmask, jnp.minimum(s, np.float32(60.0)),
                               np.float32(-1e30)))
        ve = jnp.concatenate([v[:, sl], ones_col], axis=1)
        oh = _dot(es, ve, _PREC_FAST)
        outs.append(oh[:, :HD] * (1.0 / oh[:, HD:HD + 1]))
    o = jnp.concatenate(outs, axis=1)
    return h + _dot(o, wo, _PREC_FAST) + bo


def _moe(h, g, b, gwh_hi, gwh_lo, gbt, ew1, eb1, ew2, eb2, uw1, ub1, uw2,
         ub2):
    h2 = _ln(h, g, b)
    logits = _dot3(h2, gwh_hi, gwh_lo) + gbt
    m1 = jnp.max(logits, axis=-1, keepdims=True)
    l2 = jnp.where(logits == m1, np.float32(-1e30), logits)
    m2 = jnp.max(l2, axis=-1, keepdims=True)
    keep = logits >= m2
    ex = jnp.where(keep, jnp.exp(logits - m1), 0.0)
    denom = jnp.sum(ex, axis=-1, keepdims=True)
    gates = ex / denom
    omega = 1.0 - 1.0 / denom
    acc = jnp.zeros((NPAD, D), jnp.float32)
    for ei in range(E):
        t1 = _gelu(_dot(h2, ew1[ei], _PREC_FAST) + eb1[ei])
        t2 = _dot(t1, ew2[ei], _PREC_FAST) + eb2[ei]
        acc = acc + gates[:, ei:ei + 1] * t2
    u = _gelu(_dot(h2, uw1, _PREC_FAST) + ub1)
    u = _dot(u, uw2, _PREC_FAST) + ub2
    return h + acc + omega * u, logits


def _head_body(cls_ref, g_ref, b_ref, hw_ref, hb_ref, tid_ref, fold_ref,
               out_ref):
    c = _ln(cls_ref[...], g_ref[...], b_ref[...])
    all_logits = _dot(c, hw_ref[...]) + hb_ref[...]
    col = lax.broadcasted_iota(jnp.int32, (B, T * 2), 1)
    sel = (col // 2) == tid_ref[...]
    masked = jnp.where(sel, all_logits, 0.0)
    out_ref[...] = _dot(masked, fold_ref[...])


def _fwd_body(ids_ref, x_ref, acosh_ref, acosl_ref, asinh_ref, asinl_ref, pwh_ref, pwl_ref, pb_ref, png_ref,
              pnb_ref, clsp_ref, pos_ref, ln1g_ref, ln1b_ref, wq_ref, wk_ref,
              wv_ref, bq_ref, bk_ref, bv_ref, wo_ref, bo_ref, ln2g_ref,
              ln2b_ref, gwhh_ref, gwhl_ref, gbt_ref, ew1_ref, eb1_ref,
              ew2_ref, eb2_ref,
              uw1_ref, ub1_ref, uw2_ref, ub2_ref,
              logits_ref, cls_ref):
    kmask = lax.broadcasted_iota(jnp.int32, (NPAD, NPAD), 1) < NTOK
    x = x_ref[0]
    x_hi = _trunc_bf16(x)
    x_lo = x - x_hi
    re = (_dot(x_hi, acosh_ref[...], _PREC_FAST)
          + _dot(x_hi, acosl_ref[...], _PREC_FAST)
          + _dot(x_lo, acosh_ref[...], _PREC_FAST))
    im = (_dot(x_hi, asinh_ref[...], _PREC_FAST)
          + _dot(x_hi, asinl_ref[...], _PREC_FAST)
          + _dot(x_lo, asinh_ref[...], _PREC_FAST))
    mag = jnp.sqrt(re * re + im * im)
    t = _dot3(mag, pwh_ref[...], pwl_ref[...]) + pb_ref[...]
    t = _ln(t, png_ref[...], pnb_ref[...])
    t = _gelu(t)
    t = t + pos_ref[...]
    h = jnp.concatenate(
        [clsp_ref[...], t, jnp.zeros((NPAD - NTOK, D), jnp.float32)], axis=0)

    for l in range(NL):
        h = _attn(h, ln1g_ref[l:l + 1], ln1b_ref[l:l + 1], wq_ref[l],
                  wk_ref[l], wv_ref[l], bq_ref[l:l + 1], bk_ref[l:l + 1],
                  bv_ref[l:l + 1], wo_ref[l], bo_ref[l:l + 1], kmask)
        h, logits = _moe(h, ln2g_ref[l:l + 1], ln2b_ref[l:l + 1],
                         gwhh_ref[l], gwhl_ref[l],
                         gbt_ref[0, l:l + 1], ew1_ref[l],
                         eb1_ref[l],
                         ew2_ref[l], eb2_ref[l], uw1_ref[l],
                         ub1_ref[l, 0:1], uw2_ref[l], ub2_ref[l, 0:1])
        logits_ref[0, l] = logits

    cls_ref[0] = h[0:1, :]


def kernel(x_dict, task_ids, proj_W, proj_b, pln_g, pln_b, cls_token,
           pos_embed, ln1_g, ln1_b, Wq, Wk, Wv, bq, bk, bv, Wo, bo, ln2_g,
           ln2_b, task_embed, gate_W, gate_b, eW1, eb1, eW2, eb2, uW1, ub1,
           uW2, ub2, fln_g, fln_b, head_W, head_b):
    task_ids = task_ids.astype(jnp.int32)
    xseg = x_dict.reshape(B, C * SEGS, SEG_LEN)
    cls_pos0 = (cls_token.reshape(1, D) + pos_embed[0, 0].reshape(1, D))
    pw_hi, pw_lo = _split_hi_lo(proj_W)
    pos_rest = pos_embed[0, 1:NTOK]

    gwh_hi, gwh_lo = _split_hi_lo(gate_W[:, :D, :])
    gbt = (jnp.einsum('lte,leo->tlo', task_embed, gate_W[:, D:, :],
                      precision=lax.Precision.HIGHEST)
           + gate_b[None, :, :])

    full = lambda shp: pl.BlockSpec(shp, lambda i, ids: (0,) * len(shp))
    grid_spec = pltpu.PrefetchScalarGridSpec(
        num_scalar_prefetch=1,
        grid=(B,),
        in_specs=[
            pl.BlockSpec((1, C * SEGS, SEG_LEN), lambda i, ids: (i, 0, 0)),
            full((SEG_LEN, FLAT)), full((SEG_LEN, FLAT)),
            full((SEG_LEN, FLAT)), full((SEG_LEN, FLAT)),
            full((FLAT, D)), full((FLAT, D)), full((1, D)), full((1, D)), full((1, D)),
            full((1, D)), full((C * SEGS, D)),
            full((NL, D)), full((NL, D)),
            full((NL, D, D)), full((NL, D, D)), full((NL, D, D)),
            full((NL, D)), full((NL, D)), full((NL, D)),
            full((NL, D, D)), full((NL, D)),
            full((NL, D)), full((NL, D)),
            full((NL, D, E)), full((NL, D, E)),
            pl.BlockSpec((1, NL, E), lambda i, ids: (ids[i], 0, 0)),
            full((NL, E, D, DFF)), full((NL, E, DFF)),
            full((NL, E, DFF, D)), full((NL, E, D)),
            full((NL, D, DFF)), full((NL, 1, DFF)),
            full((NL, DFF, D)), full((NL, 1, D)),
        ],
        out_specs=[
            pl.BlockSpec((1, NL, NPAD, E), lambda i, ids: (i, 0, 0, 0)),
            pl.BlockSpec((1, 1, D), lambda i, ids: (i, 0, 0)),
        ],
    )
    logits_all, cls_rows = pl.pallas_call(
        _fwd_body,
        grid_spec=grid_spec,
        out_shape=[jax.ShapeDtypeStruct((B, NL, NPAD, E), jnp.float32),
                   jax.ShapeDtypeStruct((B, 1, D), jnp.float32)],
    )(task_ids, xseg, _A_COS_HI, _A_COS_LO, _A_SIN_HI, _A_SIN_LO, pw_hi, pw_lo, proj_b.reshape(1, D),
      pln_g.reshape(1, D), pln_b.reshape(1, D), cls_pos0, pos_rest,
      ln1_g, ln1_b, Wq, Wk, Wv, bq, bk, bv, Wo, bo, ln2_g, ln2_b,
      gwh_hi, gwh_lo, gbt, eW1, eb1, eW2, eb2, uW1, ub1.reshape(NL, 1, DFF),
      uW2, ub2.reshape(NL, 1, D))

    hw_flat = jnp.transpose(head_W, (1, 0, 2)).reshape(D, T * 2)
    hb_flat = head_b.reshape(1, T * 2)
    fold = np.zeros((T * 2, 2), np.float32)
    fold[0::2, 0] = 1.0
    fold[1::2, 1] = 1.0
    task_logits = pl.pallas_call(
        _head_body,
        in_specs=[pl.BlockSpec((B, D), lambda: (0, 0)),
                  pl.BlockSpec((1, D), lambda: (0, 0)),
                  pl.BlockSpec((1, D), lambda: (0, 0)),
                  pl.BlockSpec((D, T * 2), lambda: (0, 0)),
                  pl.BlockSpec((1, T * 2), lambda: (0, 0)),
                  pl.BlockSpec((B, 1), lambda: (0, 0)),
                  pl.BlockSpec((T * 2, 2), lambda: (0, 0))],
        out_specs=pl.BlockSpec((B, 2), lambda: (0, 0)),
        out_shape=jax.ShapeDtypeStruct((B, 2), jnp.float32),
    )(cls_rows.reshape(B, D), fln_g.reshape(1, D), fln_b.reshape(1, D),
      hw_flat, hb_flat, task_ids.reshape(B, 1), fold)

    final_router = logits_all[:, :, :NTOK, :].reshape(-1, E)
    return (task_logits, final_router)
